# Initial kernel scaffold; baseline (speedup 1.0000x reference)
import jax
import jax.numpy as jnp
from jax.experimental import pallas as pl


def kernel(pos, batch, sW1, sb1, sW2, sb2, sW3, sb3, sF1, sfb1, sF2, sfb2, sF3, sfb3, c1W1, c1b1, c1W2, c1b2, c1W3, c1b3, c2W1, c2b1, lW, lb, mW1, mb1, mW2, mb2, mW3, mb3):
    raise NotImplementedError("write your pallas kernel here")


# speedup vs baseline: 20.6094x; 20.6094x over previous
import jax
import jax.numpy as jnp
from jax.experimental import pallas as pl

_BN_S = (1.0 + 1e-5) ** -0.5
_P = 1024
_K = 4


def _dense_bn(a, w, b):
    return jnp.maximum(jnp.dot(a, w, preferred_element_type=jnp.float32) + b, 0.0) * _BN_S


def _knn4(x):
    f32 = jnp.float32
    g = jax.lax.dot_general(x, x, (((1,), (1,)), ((), ())),
                            preferred_element_type=f32)
    rows = jax.lax.broadcasted_iota(jnp.int32, (_P, _P), 0)
    cols = jax.lax.broadcasted_iota(jnp.int32, (_P, _P), 1)
    d2r = jnp.max(jnp.where(rows == cols, g, -jnp.inf), axis=0, keepdims=True)
    r = d2r - 2.0 * g
    idxs = []
    for _ in range(_K):
        m = jnp.min(r, axis=1, keepdims=True)
        ik = jnp.min(jnp.where(r == m, cols, _P), axis=1, keepdims=True)
        idxs.append(ik)
        r = jnp.where(cols == ik, jnp.inf, r)
    return idxs, cols


def _edge_conv(x, idxs, cols, wa_t, wb_t, b1, tail):
    base = jnp.dot(x, wa_t, preferred_element_type=jnp.float32) + b1
    acc = None
    for ik in idxs:
        oh = (cols == ik).astype(jnp.float32)
        xj = jnp.dot(oh, x, preferred_element_type=jnp.float32)
        h = jnp.maximum(
            base + jnp.dot(xj - x, wb_t, preferred_element_type=jnp.float32),
            0.0) * _BN_S
        for (w_t, b) in tail:
            h = _dense_bn(h, w_t, b)
        acc = h if acc is None else jnp.maximum(acc, h)
    return acc


def _net_body(pos_ref,
              sW1t, sb1, sW2t, sb2, sW3t, sb3,
              sF1t, sfb1, sF2t, sfb2, sF3t, sfb3e,
              c1At, c1Bt, c1b1, c1W2t, c1b2, c1W3t, c1b3,
              c2At, c2Bt, c2b1,
              lAt, lBt, lb,
              mW1t, mb1, mW2t, mb2, mW3t, mb3,
              out_ref):
    f32 = jnp.float32
    pos = pos_ref[0]

    t = _dense_bn(pos, sW1t[...], sb1[...])
    t = _dense_bn(t, sW2t[...], sb2[...])
    t = _dense_bn(t, sW3t[...], sb3[...])
    g = jnp.max(t, axis=0, keepdims=True)
    g = _dense_bn(g, sF1t[...], sfb1[...])
    g = _dense_bn(g, sF2t[...], sfb2[...])
    t9 = jnp.dot(g, sF3t[...], preferred_element_type=f32) + sfb3e[...]

    x = (pos[:, 0:1] * t9[:, 0:3]
         + pos[:, 1:2] * t9[:, 3:6]
         + pos[:, 2:3] * t9[:, 6:9])

    idxs, cols = _knn4(x)
    x1 = _edge_conv(x, idxs, cols, c1At[...], c1Bt[...], c1b1[...],
                    [(c1W2t[...], c1b2[...]), (c1W3t[...], c1b3[...])])

    idxs2, cols2 = _knn4(x1)
    x2 = _edge_conv(x1, idxs2, cols2, c2At[...], c2Bt[...], c2b1[...], [])

    h = jnp.maximum(
        jnp.dot(x1, lAt[...], preferred_element_type=f32)
        + jnp.dot(x2, lBt[...], preferred_element_type=f32) + lb[...],
        0.0) * _BN_S
    gg = jnp.max(h, axis=0, keepdims=True)

    m = _dense_bn(gg, mW1t[...], mb1[...])
    m = _dense_bn(m, mW2t[...], mb2[...])
    logits = jnp.dot(m, mW3t[...], preferred_element_type=f32) + mb3[...]
    z = logits - jnp.max(logits, axis=1, keepdims=True)
    out_ref[0] = z - jnp.log(jnp.sum(jnp.exp(z), axis=1, keepdims=True))


def kernel(pos, batch, sW1, sb1, sW2, sb2, sW3, sb3, sF1, sfb1, sF2, sfb2, sF3, sfb3, c1W1, c1b1, c1W2, c1b2, c1W3, c1b3, c2W1, c2b1, lW, lb, mW1, mb1, mW2, mb2, mW3, mb3):
    f32 = jnp.float32
    nb = pos.shape[0] // _P
    pos3 = pos.astype(f32).reshape(nb, _P, 3)
    eye9 = jnp.eye(3, dtype=f32).reshape(9)

    weights = (
        sW1.T, sb1[None], sW2.T, sb2[None], sW3.T, sb3[None],
        sF1.T, sfb1[None], sF2.T, sfb2[None], sF3.T, (sfb3 + eye9)[None],
        c1W1[:, :3].T, c1W1[:, 3:].T, c1b1[None], c1W2.T, c1b2[None],
        c1W3.T, c1b3[None],
        c2W1[:, :64].T, c2W1[:, 64:].T, c2b1[None],
        lW[:, :64].T, lW[:, 64:].T, lb[None],
        mW1.T, mb1[None], mW2.T, mb2[None], mW3.T, mb3[None],
    )

    w_specs = [pl.BlockSpec(w.shape, lambda b, n=w.ndim: (0,) * n)
               for w in weights]
    out = pl.pallas_call(
        _net_body,
        grid=(nb,),
        in_specs=[pl.BlockSpec((1, _P, 3), lambda b: (b, 0, 0))] + w_specs,
        out_specs=pl.BlockSpec((1, 1, 40), lambda b: (b, 0, 0)),
        out_shape=jax.ShapeDtypeStruct((nb, 1, 40), f32),
    )(pos3, *weights)
    return out.reshape(nb, 40)


# speedup vs baseline: 21.3541x; 1.0361x over previous
import jax
import jax.numpy as jnp
from jax.experimental import pallas as pl

_BN_S = (1.0 + 1e-5) ** -0.5
_P = 1024
_K = 4


def _dense_bn(a, w, b):
    return jnp.maximum(jnp.dot(a, w, preferred_element_type=jnp.float32) + b, 0.0) * _BN_S


def _knn4_onehots(x, rows, cols):
    f32 = jnp.float32
    g = jax.lax.dot_general(x, x, (((1,), (1,)), ((), ())),
                            preferred_element_type=f32)
    d2c = jnp.sum(x * x, axis=1, keepdims=True)
    d2r = jnp.max(jnp.where(rows == cols, g, -jnp.inf), axis=0, keepdims=True)
    d = jnp.maximum(d2c + d2r - 2.0 * g, 0.0)
    keys = (jax.lax.bitcast_convert_type(d, jnp.int32) & ~1023) | cols
    onehots = []
    for _ in range(_K):
        m = jnp.min(keys, axis=1, keepdims=True)
        eq = keys == m
        onehots.append(jnp.where(eq, 1.0, 0.0).astype(f32))
        keys = jnp.where(eq, jnp.iinfo(jnp.int32).max, keys)
    return onehots


def _edge_conv(x, onehots, wa_t, wb_t, b1, tail):
    base = jnp.dot(x, wa_t, preferred_element_type=jnp.float32) + b1
    acc = None
    for oh in onehots:
        xj = jnp.dot(oh, x, preferred_element_type=jnp.float32)
        h = jnp.maximum(
            base + jnp.dot(xj - x, wb_t, preferred_element_type=jnp.float32),
            0.0) * _BN_S
        for (w_t, b) in tail:
            h = _dense_bn(h, w_t, b)
        acc = h if acc is None else jnp.maximum(acc, h)
    return acc


def _net_body(pos_ref,
              sW1t, sb1, sW2t, sb2, sW3t, sb3,
              sF1t, sfb1, sF2t, sfb2, sF3t, sfb3e,
              c1At, c1Bt, c1b1, c1W2t, c1b2, c1W3t, c1b3,
              c2At, c2Bt, c2b1,
              lAt, lBt, lb,
              mW1t, mb1, mW2t, mb2, mW3t, mb3,
              out_ref):
    f32 = jnp.float32
    pos = pos_ref[0]

    t = _dense_bn(pos, sW1t[...], sb1[...])
    t = _dense_bn(t, sW2t[...], sb2[...])
    t = _dense_bn(t, sW3t[...], sb3[...])
    g = jnp.max(t, axis=0, keepdims=True)
    g = _dense_bn(g, sF1t[...], sfb1[...])
    g = _dense_bn(g, sF2t[...], sfb2[...])
    t9 = jnp.dot(g, sF3t[...], preferred_element_type=f32) + sfb3e[...]

    x = (pos[:, 0:1] * t9[:, 0:3]
         + pos[:, 1:2] * t9[:, 3:6]
         + pos[:, 2:3] * t9[:, 6:9])

    rows = jax.lax.broadcasted_iota(jnp.int32, (_P, _P), 0)
    cols = jax.lax.broadcasted_iota(jnp.int32, (_P, _P), 1)

    oh1 = _knn4_onehots(x, rows, cols)
    x1 = _edge_conv(x, oh1, c1At[...], c1Bt[...], c1b1[...],
                    [(c1W2t[...], c1b2[...]), (c1W3t[...], c1b3[...])])

    oh2 = _knn4_onehots(x1, rows, cols)
    x2 = _edge_conv(x1, oh2, c2At[...], c2Bt[...], c2b1[...], [])

    h = jnp.maximum(
        jnp.dot(x1, lAt[...], preferred_element_type=f32)
        + jnp.dot(x2, lBt[...], preferred_element_type=f32) + lb[...],
        0.0) * _BN_S
    gg = jnp.max(h, axis=0, keepdims=True)

    m = _dense_bn(gg, mW1t[...], mb1[...])
    m = _dense_bn(m, mW2t[...], mb2[...])
    logits = jnp.dot(m, mW3t[...], preferred_element_type=f32) + mb3[...]
    z = logits - jnp.max(logits, axis=1, keepdims=True)
    out_ref[0] = z - jnp.log(jnp.sum(jnp.exp(z), axis=1, keepdims=True))


def kernel(pos, batch, sW1, sb1, sW2, sb2, sW3, sb3, sF1, sfb1, sF2, sfb2, sF3, sfb3, c1W1, c1b1, c1W2, c1b2, c1W3, c1b3, c2W1, c2b1, lW, lb, mW1, mb1, mW2, mb2, mW3, mb3):
    f32 = jnp.float32
    nb = pos.shape[0] // _P
    pos3 = pos.astype(f32).reshape(nb, _P, 3)
    eye9 = jnp.eye(3, dtype=f32).reshape(9)

    weights = (
        sW1.T, sb1[None], sW2.T, sb2[None], sW3.T, sb3[None],
        sF1.T, sfb1[None], sF2.T, sfb2[None], sF3.T, (sfb3 + eye9)[None],
        c1W1[:, :3].T, c1W1[:, 3:].T, c1b1[None], c1W2.T, c1b2[None],
        c1W3.T, c1b3[None],
        c2W1[:, :64].T, c2W1[:, 64:].T, c2b1[None],
        lW[:, :64].T, lW[:, 64:].T, lb[None],
        mW1.T, mb1[None], mW2.T, mb2[None], mW3.T, mb3[None],
    )

    w_specs = [pl.BlockSpec(w.shape, lambda b, n=w.ndim: (0,) * n)
               for w in weights]
    out = pl.pallas_call(
        _net_body,
        grid=(nb,),
        in_specs=[pl.BlockSpec((1, _P, 3), lambda b: (b, 0, 0))] + w_specs,
        out_specs=pl.BlockSpec((1, 1, 40), lambda b: (b, 0, 0)),
        out_shape=jax.ShapeDtypeStruct((nb, 1, 40), f32),
    )(pos3, *weights)
    return out.reshape(nb, 40)


# speedup vs baseline: 22.9663x; 1.0755x over previous
import jax
import jax.numpy as jnp
from jax.experimental import pallas as pl

_BN_S = (1.0 + 1e-5) ** -0.5
_P = 1024
_K = 4


def _dense_bn(a, w, b):
    return jnp.maximum(jnp.dot(a, w, preferred_element_type=jnp.float32) + b, 0.0) * _BN_S


def _knn4_onehots(x, rows, cols):
    f32 = jnp.float32
    g = jax.lax.dot_general(x, x, (((1,), (1,)), ((), ())),
                            preferred_element_type=f32)
    d2c = jnp.sum(x * x, axis=1, keepdims=True)
    d2r = jnp.max(jnp.where(rows == cols, g, -jnp.inf), axis=0, keepdims=True)
    d = jnp.maximum(d2c + d2r - 2.0 * g, 1.17549435e-38)
    keys = (jax.lax.bitcast_convert_type(d, jnp.int32) & ~1023) | cols
    keys = jax.lax.bitcast_convert_type(keys, f32)
    onehots = []
    for _ in range(_K):
        m = jnp.min(keys, axis=1, keepdims=True)
        eq = keys == m
        onehots.append(jnp.where(eq, 1.0, 0.0).astype(f32))
        keys = jnp.where(eq, jnp.inf, keys)
    return onehots


def _edge_conv(x, onehots, wa_t, wb_t, b1, tail, gather_bf16=False):
    base = jnp.dot(x, wa_t, preferred_element_type=jnp.float32) + b1
    xg = x.astype(jnp.bfloat16) if gather_bf16 else x
    acc = None
    for oh in onehots:
        xj = jnp.dot(oh.astype(xg.dtype), xg,
                     preferred_element_type=jnp.float32)
        h = jnp.maximum(
            base + jnp.dot(xj - x, wb_t, preferred_element_type=jnp.float32),
            0.0) * _BN_S
        for (w_t, b) in tail:
            h = _dense_bn(h, w_t, b)
        acc = h if acc is None else jnp.maximum(acc, h)
    return acc


def _net_body(pos_ref,
              sW1t, sb1, sW2t, sb2, sW3t, sb3,
              sF1t, sfb1, sF2t, sfb2, sF3t, sfb3e,
              c1At, c1Bt, c1b1, c1W2t, c1b2, c1W3t, c1b3,
              c2At, c2Bt, c2b1,
              lAt, lBt, lb,
              mW1t, mb1, mW2t, mb2, mW3t, mb3,
              out_ref):
    f32 = jnp.float32
    pos = pos_ref[0]

    t = _dense_bn(pos, sW1t[...], sb1[...])
    t = _dense_bn(t, sW2t[...], sb2[...])
    t = _dense_bn(t, sW3t[...], sb3[...])
    g = jnp.max(t, axis=0, keepdims=True)
    g = _dense_bn(g, sF1t[...], sfb1[...])
    g = _dense_bn(g, sF2t[...], sfb2[...])
    t9 = jnp.dot(g, sF3t[...], preferred_element_type=f32) + sfb3e[...]

    x = (pos[:, 0:1] * t9[:, 0:3]
         + pos[:, 1:2] * t9[:, 3:6]
         + pos[:, 2:3] * t9[:, 6:9])

    rows = jax.lax.broadcasted_iota(jnp.int32, (_P, _P), 0)
    cols = jax.lax.broadcasted_iota(jnp.int32, (_P, _P), 1)

    oh1 = _knn4_onehots(x, rows, cols)
    x1 = _edge_conv(x, oh1, c1At[...], c1Bt[...], c1b1[...],
                    [(c1W2t[...], c1b2[...]), (c1W3t[...], c1b3[...])])

    oh2 = _knn4_onehots(x1, rows, cols)
    x2 = _edge_conv(x1, oh2, c2At[...], c2Bt[...], c2b1[...], [],
                    gather_bf16=True)

    h = jnp.maximum(
        jnp.dot(x1, lAt[...], preferred_element_type=f32)
        + jnp.dot(x2, lBt[...], preferred_element_type=f32) + lb[...],
        0.0) * _BN_S
    gg = jnp.max(h, axis=0, keepdims=True)

    m = _dense_bn(gg, mW1t[...], mb1[...])
    m = _dense_bn(m, mW2t[...], mb2[...])
    logits = jnp.dot(m, mW3t[...], preferred_element_type=f32) + mb3[...]
    z = logits - jnp.max(logits, axis=1, keepdims=True)
    out_ref[0] = z - jnp.log(jnp.sum(jnp.exp(z), axis=1, keepdims=True))


def kernel(pos, batch, sW1, sb1, sW2, sb2, sW3, sb3, sF1, sfb1, sF2, sfb2, sF3, sfb3, c1W1, c1b1, c1W2, c1b2, c1W3, c1b3, c2W1, c2b1, lW, lb, mW1, mb1, mW2, mb2, mW3, mb3):
    f32 = jnp.float32
    nb = pos.shape[0] // _P
    pos3 = pos.astype(f32).reshape(nb, _P, 3)
    eye9 = jnp.eye(3, dtype=f32).reshape(9)

    weights = (
        sW1.T, sb1[None], sW2.T, sb2[None], sW3.T, sb3[None],
        sF1.T, sfb1[None], sF2.T, sfb2[None], sF3.T, (sfb3 + eye9)[None],
        c1W1[:, :3].T, c1W1[:, 3:].T, c1b1[None], c1W2.T, c1b2[None],
        c1W3.T, c1b3[None],
        c2W1[:, :64].T, c2W1[:, 64:].T, c2b1[None],
        lW[:, :64].T, lW[:, 64:].T, lb[None],
        mW1.T, mb1[None], mW2.T, mb2[None], mW3.T, mb3[None],
    )

    w_specs = [pl.BlockSpec(w.shape, lambda b, n=w.ndim: (0,) * n)
               for w in weights]
    out = pl.pallas_call(
        _net_body,
        grid=(nb,),
        in_specs=[pl.BlockSpec((1, _P, 3), lambda b: (b, 0, 0))] + w_specs,
        out_specs=pl.BlockSpec((1, 1, 40), lambda b: (b, 0, 0)),
        out_shape=jax.ShapeDtypeStruct((nb, 1, 40), f32),
    )(pos3, *weights)
    return out.reshape(nb, 40)
